# baseline (speedup 1.0000x reference)
import jax
import jax.numpy as jnp
from jax.experimental import pallas as pl
from jax.experimental.pallas import tpu as pltpu

MODEL_DIM = 768
DIM_FF = 2048
NUM_EXPERTS = 8
TOP_K = 2

TOKEN_BLOCK = 256


def _moe_body(xf_ref, gw_ref, gb_ref, w1_ref, b1_ref, w2_ref, b2_ref, out_ref):
    e = pl.program_id(1)
    x = xf_ref[...]
    logits = jnp.dot(x, gw_ref[...], preferred_element_type=jnp.float32)
    logits = logits + gb_ref[...]

    iota_e = jax.lax.broadcasted_iota(jnp.int32, logits.shape, 1)
    v1 = jnp.max(logits, axis=-1, keepdims=True)
    i1 = jnp.argmax(logits, axis=-1, keepdims=True).astype(jnp.int32)
    masked = jnp.where(iota_e == i1, -jnp.inf, logits)
    v2 = jnp.max(masked, axis=-1, keepdims=True)
    i2 = jnp.argmax(masked, axis=-1, keepdims=True).astype(jnp.int32)
    t = jnp.exp(v2 - v1)
    w_first = 1.0 / (1.0 + t)
    w_second = t / (1.0 + t)
    combine_e = w_first * (i1 == e) + w_second * (i2 == e)

    h = jnp.dot(x, w1_ref[0], preferred_element_type=jnp.float32) + b1_ref[0]
    h = jnp.maximum(h, 0.0)
    y = jnp.dot(h, w2_ref[0], preferred_element_type=jnp.float32) + b2_ref[0]
    contrib = combine_e * y

    @pl.when(e == 0)
    def _():
        out_ref[...] = contrib

    @pl.when(e != 0)
    def _():
        out_ref[...] = out_ref[...] + contrib


def kernel(x, gate_W, gate_b, W1, b1, W2, b2):
    batch, seq, _ = x.shape
    xf = x.reshape(-1, MODEL_DIM)
    T = xf.shape[0]
    n_tb = T // TOKEN_BLOCK
    gb2 = gate_b.reshape(1, NUM_EXPERTS)
    b1r = b1.reshape(NUM_EXPERTS, 1, DIM_FF)
    b2r = b2.reshape(NUM_EXPERTS, 1, MODEL_DIM)

    out = pl.pallas_call(
        _moe_body,
        grid=(n_tb, NUM_EXPERTS),
        in_specs=[
            pl.BlockSpec((TOKEN_BLOCK, MODEL_DIM), lambda t, e: (t, 0)),
            pl.BlockSpec((MODEL_DIM, NUM_EXPERTS), lambda t, e: (0, 0)),
            pl.BlockSpec((1, NUM_EXPERTS), lambda t, e: (0, 0)),
            pl.BlockSpec((1, MODEL_DIM, DIM_FF), lambda t, e: (e, 0, 0)),
            pl.BlockSpec((1, 1, DIM_FF), lambda t, e: (e, 0, 0)),
            pl.BlockSpec((1, DIM_FF, MODEL_DIM), lambda t, e: (e, 0, 0)),
            pl.BlockSpec((1, 1, MODEL_DIM), lambda t, e: (e, 0, 0)),
        ],
        out_specs=pl.BlockSpec((TOKEN_BLOCK, MODEL_DIM), lambda t, e: (t, 0)),
        out_shape=jax.ShapeDtypeStruct((T, MODEL_DIM), jnp.float32),
        compiler_params=pltpu.CompilerParams(
            dimension_semantics=("arbitrary", "arbitrary"),
        ),
    )(xf, gate_W, gb2, W1, b1r, W2, b2r)
    return out.reshape(batch, seq, MODEL_DIM)
